# baseline (speedup 1.0000x reference)
import math

import jax
import jax.numpy as jnp
from jax import lax
from jax.experimental import pallas as pl
from jax.experimental.pallas import tpu as pltpu

_B, _S, _NC, _NA, _V = 16, 128, 6, 16, 512
_EOS = 3
_TOL = 3
_ALPHA = 2.0
_BS = _B * _S
_M = _BS * _NA
_BPB = 2
_PBLK = _BPB * _S
_GRID = _BS // _PBLK
_SHIFT_W = [math.exp(-_ALPHA * abs(s)) for s in range(-_TOL, _TOL + 1)]
_W_TOT = sum(_SHIFT_W)
_F = [sum(math.exp(-_ALPHA * j) for j in range(k, _TOL + 1)) for k in range(_TOL + 1)]


def _body(clT_ref, cmds_ref, am_ref, lane_ref, tok_ref, x_ref,
          num_ref, den_ref, cnum_ref, cden_ref):
    i = pl.program_id(0)

    r = lax.broadcasted_iota(jnp.int32, (_S, _S), 0)
    c = lax.broadcasted_iota(jnp.int32, (_S, _S), 1)
    lower = (r <= c).astype(jnp.float32)
    eye = (r == c).astype(jnp.float32)

    @pl.when(i == 0)
    def _prep():
        cmds = cmds_ref[...]
        eos = (cmds == _EOS).astype(jnp.float32)
        cum = jnp.dot(eos, lower, preferred_element_type=jnp.float32)
        valid = (cum <= 1.0).astype(jnp.float32)

        m = clT_ref[0]
        for ci in range(1, _NC):
            m = jnp.maximum(m, clT_ref[ci])
        ssum = jnp.zeros_like(m)
        xt = jnp.zeros_like(m)
        for ci in range(_NC):
            xc = clT_ref[ci]
            ssum = ssum + jnp.exp(xc - m)
            xt = xt + jnp.where(cmds == ci, xc, 0.0)
        lse_c = m + jnp.log(ssum)
        closs = lse_c - xt
        closs = jnp.where(jnp.isnan(closs), 0.0, closs)
        cnum_ref[0, 0] = jnp.sum(closs * valid)
        cden_ref[0, 0] = jnp.sum(valid)
        num_ref[0, 0] = jnp.float32(0.0)
        den_ref[0, 0] = jnp.float32(0.0)

    crows = cmds_ref[pl.ds(i * _BPB, _BPB), :]
    eos_rows = (crows == _EOS).astype(jnp.float32)
    cum_col = lax.dot_general(lower, eos_rows, (((0,), (1,)), ((), ())),
                              preferred_element_type=jnp.float32)
    valid_col = (cum_col <= 1.0).astype(jnp.float32)
    cmdf_rows = crows.astype(jnp.float32)
    cmd_col = lax.dot_general(eye, cmdf_rows, (((1,), (1,)), ((), ())),
                              preferred_element_type=jnp.float32)

    x = x_ref[...]
    m = jnp.max(x, axis=2, keepdims=True)
    e = jnp.exp(x - m)
    ssum = jnp.sum(e, axis=2, keepdims=True)
    lse = m + jnp.log(ssum)

    tok = jnp.clip(tok_ref[...], 0, _V - 1)
    tf = tok.astype(jnp.float32)[..., None]
    lane = lane_ref[...].reshape(1, 1, _V)
    ad = jnp.abs(lane - tf)
    w = jnp.exp(jnp.float32(-_ALPHA) * ad)
    g = jnp.sum(w * x, axis=2, keepdims=True)

    c0 = jnp.where(tok == 0, jnp.float32(_F[1]),
         jnp.where(tok == 1, jnp.float32(_F[2]),
         jnp.where(tok == 2, jnp.float32(_F[3]), jnp.float32(0.0))))
    tv = (_V - 1) - tok
    c1 = jnp.where(tv == 0, jnp.float32(_F[1]),
         jnp.where(tv == 1, jnp.float32(_F[2]),
         jnp.where(tv == 2, jnp.float32(_F[3]), jnp.float32(0.0))))
    g = g + c0[..., None] * x[:, :, 0:1] + c1[..., None] * x[:, :, _V - 1:_V]

    loss = (jnp.float32(_W_TOT) * lse - g) * jnp.float32(1.0 / (_W_TOT + 1e-8))
    loss = jnp.where(jnp.isnan(loss), 0.0, loss)

    nacc = jnp.float32(0.0)
    dacc = jnp.float32(0.0)
    for bb in range(_BPB):
        cm = jnp.zeros((_S, _NA), jnp.float32)
        for ci in range(_NC):
            amrow = am_ref[ci:ci + 1, :]
            cm = cm + jnp.where(cmd_col[:, bb:bb + 1] == ci, 1.0, 0.0) * amrow
        wm = valid_col[:, bb:bb + 1] * cm
        nacc += jnp.sum(loss[bb * _S:(bb + 1) * _S] * wm[..., None])
        dacc += jnp.sum(wm)
    num_ref[0, 0] += nacc
    den_ref[0, 0] += dacc


def kernel(command_logits, unified_args_logits, commands, args_tokens, args_mask):
    clT = command_logits.astype(jnp.float32).transpose(2, 0, 1)
    cmds = commands.astype(jnp.int32)
    x3 = unified_args_logits.astype(jnp.float32).reshape(_BS, _NA, _V)
    tok2 = args_tokens.astype(jnp.int32).reshape(_BS, _NA)
    lanef = jnp.arange(_V, dtype=jnp.float32).reshape(1, _V)

    num, den, cnum, cden = pl.pallas_call(
        _body,
        grid=(_GRID,),
        out_shape=(
            jax.ShapeDtypeStruct((1, 1), jnp.float32),
            jax.ShapeDtypeStruct((1, 1), jnp.float32),
            jax.ShapeDtypeStruct((1, 1), jnp.float32),
            jax.ShapeDtypeStruct((1, 1), jnp.float32),
        ),
        in_specs=[
            pl.BlockSpec((_NC, _B, _S), lambda i: (0, 0, 0)),
            pl.BlockSpec((_B, _S), lambda i: (0, 0)),
            pl.BlockSpec((_NC, _NA), lambda i: (0, 0)),
            pl.BlockSpec((1, _V), lambda i: (0, 0)),
            pl.BlockSpec((_PBLK, _NA), lambda i: (i, 0)),
            pl.BlockSpec((_PBLK, _NA, _V), lambda i: (i, 0, 0)),
        ],
        out_specs=(
            pl.BlockSpec((1, 1), lambda i: (0, 0), memory_space=pltpu.SMEM),
            pl.BlockSpec((1, 1), lambda i: (0, 0), memory_space=pltpu.SMEM),
            pl.BlockSpec((1, 1), lambda i: (0, 0), memory_space=pltpu.SMEM),
            pl.BlockSpec((1, 1), lambda i: (0, 0), memory_space=pltpu.SMEM),
        ),
    )(clT, cmds, args_mask.astype(jnp.float32), lanef, tok2, x3)

    loss_cmd = cnum[0, 0] / (cden[0, 0] + 1e-8)
    den_s = den[0, 0]
    la = num[0, 0] / (den_s + 1e-8)
    loss_args = jnp.where(den_s < 1.0, jnp.float32(0.0), la)
    total = loss_cmd + loss_args
    return total, loss_cmd, loss_args


# speedup vs baseline: 1.0578x; 1.0578x over previous
import math

import jax
import jax.numpy as jnp
from jax import lax
from jax.experimental import pallas as pl
from jax.experimental.pallas import tpu as pltpu

_B, _S, _NC, _NA, _V = 16, 128, 6, 16, 512
_EOS = 3
_TOL = 3
_ALPHA = 2.0
_BS = _B * _S
_M = _BS * _NA
_BPB = 1
_PBLK = _BPB * _S
_GRID = _BS // _PBLK
_SHIFT_W = [math.exp(-_ALPHA * abs(s)) for s in range(-_TOL, _TOL + 1)]
_W_TOT = sum(_SHIFT_W)
_F = [sum(math.exp(-_ALPHA * j) for j in range(k, _TOL + 1)) for k in range(_TOL + 1)]


def _body(clT_ref, cmds_ref, am_ref, lane_ref, tok_ref, x_ref,
          num_ref, den_ref, cnum_ref, cden_ref):
    i = pl.program_id(0)

    r = lax.broadcasted_iota(jnp.int32, (_S, _S), 0)
    c = lax.broadcasted_iota(jnp.int32, (_S, _S), 1)
    lower = (r <= c).astype(jnp.float32)
    eye = (r == c).astype(jnp.float32)

    @pl.when(i == 0)
    def _prep():
        cmds = cmds_ref[...]
        eos = (cmds == _EOS).astype(jnp.float32)
        cum = jnp.dot(eos, lower, preferred_element_type=jnp.float32)
        valid = (cum <= 1.0).astype(jnp.float32)

        m = clT_ref[0]
        for ci in range(1, _NC):
            m = jnp.maximum(m, clT_ref[ci])
        ssum = jnp.zeros_like(m)
        xt = jnp.zeros_like(m)
        for ci in range(_NC):
            xc = clT_ref[ci]
            ssum = ssum + jnp.exp(xc - m)
            xt = xt + jnp.where(cmds == ci, xc, 0.0)
        lse_c = m + jnp.log(ssum)
        closs = lse_c - xt
        closs = jnp.where(jnp.isnan(closs), 0.0, closs)
        cnum_ref[0, 0] = jnp.sum(closs * valid)
        cden_ref[0, 0] = jnp.sum(valid)
        num_ref[0, 0] = jnp.float32(0.0)
        den_ref[0, 0] = jnp.float32(0.0)

    crows = cmds_ref[pl.ds(i * _BPB, _BPB), :]
    eos_rows = (crows == _EOS).astype(jnp.float32)
    cum_col = lax.dot_general(lower, eos_rows, (((0,), (1,)), ((), ())),
                              preferred_element_type=jnp.float32)
    valid_col = (cum_col <= 1.0).astype(jnp.float32)
    cmdf_rows = crows.astype(jnp.float32)
    cmd_col = lax.dot_general(eye, cmdf_rows, (((1,), (1,)), ((), ())),
                              preferred_element_type=jnp.float32)

    x = x_ref[...]
    m = jnp.max(x, axis=2, keepdims=True)
    e = jnp.exp(x - m)
    ssum = jnp.sum(e, axis=2, keepdims=True)
    lse = m + jnp.log(ssum)

    tok = jnp.clip(tok_ref[...], 0, _V - 1)
    tf = tok.astype(jnp.float32)[..., None]
    lane = lane_ref[...].reshape(1, 1, _V)
    ad = jnp.abs(lane - tf)
    w = jnp.exp(jnp.float32(-_ALPHA) * ad)
    g = jnp.sum(w * x, axis=2, keepdims=True)

    c0 = jnp.where(tok == 0, jnp.float32(_F[1]),
         jnp.where(tok == 1, jnp.float32(_F[2]),
         jnp.where(tok == 2, jnp.float32(_F[3]), jnp.float32(0.0))))
    tv = (_V - 1) - tok
    c1 = jnp.where(tv == 0, jnp.float32(_F[1]),
         jnp.where(tv == 1, jnp.float32(_F[2]),
         jnp.where(tv == 2, jnp.float32(_F[3]), jnp.float32(0.0))))
    g = g + c0[..., None] * x[:, :, 0:1] + c1[..., None] * x[:, :, _V - 1:_V]

    loss = (jnp.float32(_W_TOT) * lse - g) * jnp.float32(1.0 / (_W_TOT + 1e-8))
    loss = jnp.where(jnp.isnan(loss), 0.0, loss)

    nacc = jnp.float32(0.0)
    dacc = jnp.float32(0.0)
    for bb in range(_BPB):
        cm = jnp.zeros((_S, _NA), jnp.float32)
        for ci in range(_NC):
            amrow = am_ref[ci:ci + 1, :]
            cm = cm + jnp.where(cmd_col[:, bb:bb + 1] == ci, 1.0, 0.0) * amrow
        wm = valid_col[:, bb:bb + 1] * cm
        nacc += jnp.sum(loss[bb * _S:(bb + 1) * _S] * wm[..., None])
        dacc += jnp.sum(wm)
    num_ref[0, 0] += nacc
    den_ref[0, 0] += dacc


def kernel(command_logits, unified_args_logits, commands, args_tokens, args_mask):
    clT = command_logits.astype(jnp.float32).transpose(2, 0, 1)
    cmds = commands.astype(jnp.int32)
    x3 = unified_args_logits.astype(jnp.float32).reshape(_BS, _NA, _V)
    tok2 = args_tokens.astype(jnp.int32).reshape(_BS, _NA)
    lanef = jnp.arange(_V, dtype=jnp.float32).reshape(1, _V)

    num, den, cnum, cden = pl.pallas_call(
        _body,
        grid=(_GRID,),
        out_shape=(
            jax.ShapeDtypeStruct((1, 1), jnp.float32),
            jax.ShapeDtypeStruct((1, 1), jnp.float32),
            jax.ShapeDtypeStruct((1, 1), jnp.float32),
            jax.ShapeDtypeStruct((1, 1), jnp.float32),
        ),
        in_specs=[
            pl.BlockSpec((_NC, _B, _S), lambda i: (0, 0, 0)),
            pl.BlockSpec((_B, _S), lambda i: (0, 0)),
            pl.BlockSpec((_NC, _NA), lambda i: (0, 0)),
            pl.BlockSpec((1, _V), lambda i: (0, 0)),
            pl.BlockSpec((_PBLK, _NA), lambda i: (i, 0)),
            pl.BlockSpec((_PBLK, _NA, _V), lambda i: (i, 0, 0)),
        ],
        out_specs=(
            pl.BlockSpec((1, 1), lambda i: (0, 0), memory_space=pltpu.SMEM),
            pl.BlockSpec((1, 1), lambda i: (0, 0), memory_space=pltpu.SMEM),
            pl.BlockSpec((1, 1), lambda i: (0, 0), memory_space=pltpu.SMEM),
            pl.BlockSpec((1, 1), lambda i: (0, 0), memory_space=pltpu.SMEM),
        ),
    )(clT, cmds, args_mask.astype(jnp.float32), lanef, tok2, x3)

    loss_cmd = cnum[0, 0] / (cden[0, 0] + 1e-8)
    den_s = den[0, 0]
    la = num[0, 0] / (den_s + 1e-8)
    loss_args = jnp.where(den_s < 1.0, jnp.float32(0.0), la)
    total = loss_cmd + loss_args
    return total, loss_cmd, loss_args
